# baseline (speedup 1.0000x reference)
import functools

import jax
import jax.numpy as jnp
from jax import lax
from jax.experimental import pallas as pl
from jax.experimental.pallas import tpu as pltpu
from jax.experimental.pallas import tpu_sc as plsc

B = 1024
D = 32
S = 10240
N_ALL = B + S
KPG = 128
NW = 32
BPW = N_ALL // NW

NBLK = 1024

_F32_MAX = float(jnp.finfo(jnp.float32).max)
_F32_TINY = float(jnp.finfo(jnp.float32).tiny)


def _sc_fetch_blocks(table_t, ids):
    mesh = plsc.VectorSubcoreMesh(core_axis_name="c", subcore_axis_name="s")

    @functools.partial(
        pl.kernel,
        mesh=mesh,
        out_type=jax.ShapeDtypeStruct((D, 128 * N_ALL), jnp.float32),
        scratch_types=[
            pltpu.VMEM((BPW,), jnp.int32),
            pltpu.SemaphoreType.DMA,
        ],
    )
    def fetch_kernel(tt_hbm, ids_hbm, staged_out, idx_v, sem):
        wid = lax.axis_index("s") * 2 + lax.axis_index("c")
        base = wid * BPW
        pltpu.sync_copy(ids_hbm.at[pl.ds(base, BPW)], idx_v)

        @pl.loop(0, BPW // 16)
        def _bursts(g):
            ids16 = idx_v[pl.ds(g * 16, 16)]
            cps = []
            for k in range(16):
                src = pl.multiple_of((ids16[k] >> 7) * 128, 128)
                dst = pl.multiple_of((base + g * 16 + k) * 128, 128)
                cps.append(pltpu.async_copy(
                    tt_hbm.at[:, pl.ds(src, 128)],
                    staged_out.at[:, pl.ds(dst, 128)],
                    sem))
            for cp in cps:
                cp.wait()

    return fetch_kernel(table_t, ids)


def _extract_body(ids_ref, staged_ref, out_ref):
    g = pl.program_id(0)
    lane = jax.lax.broadcasted_iota(jnp.int32, (1, 128), 1)
    out = jnp.zeros((D, KPG), jnp.float32)
    for j in range(KPG):
        c = ids_ref[g * KPG + j] & 127
        rolled = pltpu.roll(staged_ref[:, j * 128:(j + 1) * 128], j - c,
                            axis=1)
        out = jnp.where(lane == j, rolled, out)
    off = pl.multiple_of(g * KPG, KPG)
    out_ref[:, pl.ds(off, KPG)] = out


def _tc_gather(table_t, ids):
    staged = _sc_fetch_blocks(table_t, ids)
    grid_spec = pltpu.PrefetchScalarGridSpec(
        num_scalar_prefetch=1,
        grid=(N_ALL // KPG,),
        in_specs=[pl.BlockSpec((D, KPG * 128), lambda g, ids_ref: (0, g))],
        out_specs=pl.BlockSpec((D, N_ALL), lambda g, ids_ref: (0, 0)),
    )
    return pl.pallas_call(
        _extract_body,
        grid_spec=grid_spec,
        out_shape=jax.ShapeDtypeStruct((D, N_ALL), jnp.float32),
    )(ids, staged)


def _sc_gather_bias(bias, ids):
    mesh = plsc.VectorSubcoreMesh(core_axis_name="c", subcore_axis_name="s")

    @functools.partial(
        pl.kernel,
        mesh=mesh,
        compiler_params=pltpu.CompilerParams(use_tc_tiling_on_sc=False),
        out_type=jax.ShapeDtypeStruct((N_ALL,), jnp.float32),
        scratch_types=[
            pltpu.VMEM((BPW,), jnp.int32),
            pltpu.VMEM((BPW,), jnp.float32),
            pltpu.SemaphoreType.DMA,
        ],
    )
    def gather_kernel(bias_hbm, ids_hbm, bias_out, idx_v, bias_v, sem):
        wid = lax.axis_index("s") * 2 + lax.axis_index("c")
        base = wid * BPW
        pltpu.sync_copy(ids_hbm.at[pl.ds(base, BPW)], idx_v)
        pltpu.async_copy(bias_hbm.at[idx_v], bias_v, sem).wait()
        pltpu.sync_copy(bias_v, bias_out.at[pl.ds(base, BPW)])

    return gather_kernel(bias, ids)


def _expected_counts(p, nt):
    acc = jnp.ones_like(p)
    cur = 1.0 - p
    for k in range(32):
        bit = ((nt >> k) & 1).astype(jnp.float32)
        acc = acc * (cur * bit + (1.0 - bit))
        cur = cur * cur
    eq = (nt == S).astype(jnp.float32)
    expected = (p * jnp.float32(S)) * eq + (1.0 - acc) * (1.0 - eq)
    return jnp.maximum(expected, _F32_TINY)


def _flash_body(nt_ref, usert_ref, pos_embt_ref, samp_embt_ref, pos_bias_ref,
                samp_bias_ref, pos_ids_ref, samp_ids_ref, p_pos_ref,
                p_samp_ref, out_ref, m_ref, s_ref, pos_ref):
    j = pl.program_id(0)
    nt = nt_ref[0, 0]

    @pl.when(j == 0)
    def _init():
        e_p = _expected_counts(p_pos_ref[...], nt)
        pos = (jnp.sum(usert_ref[...] * pos_embt_ref[...], axis=0,
                       keepdims=True)
               + pos_bias_ref[...] - jnp.log(e_p))
        pos_ref[...] = pos
        m_ref[...] = pos
        s_ref[...] = jnp.ones_like(pos)

    z = lax.dot_general(samp_embt_ref[...], usert_ref[...],
                        (((0,), (0,)), ((), ())),
                        preferred_element_type=jnp.float32)
    z = z + samp_bias_ref[...]
    hit = samp_ids_ref[...] == pos_ids_ref[...]
    z = jnp.where(hit, -_F32_MAX, z)
    e_s = _expected_counts(p_samp_ref[...], nt)
    z = z - jnp.log(e_s)

    m_old = m_ref[...]
    m_new = jnp.maximum(m_old, jnp.max(z, axis=0, keepdims=True))
    s_ref[...] = (s_ref[...] * jnp.exp(m_old - m_new)
                  + jnp.sum(jnp.exp(z - m_new), axis=0, keepdims=True))
    m_ref[...] = m_new

    @pl.when(j == pl.num_programs(0) - 1)
    def _fin():
        lse = m_ref[...] + jnp.log(s_ref[...])
        out_ref[0, 0] = jnp.sum(lse - pos_ref[...]) / jnp.float32(B)


def _fused_loss(nt2, usert, embt, pos_bias2, samp_bias2, pos_ids2, samp_ids2,
                p_pos2, p_samp2):
    return pl.pallas_call(
        _flash_body,
        grid=(S // NBLK,),
        in_specs=[
            pl.BlockSpec(memory_space=pltpu.SMEM),
            pl.BlockSpec((D, B), lambda j: (0, 0)),
            pl.BlockSpec((D, B), lambda j: (0, 0)),
            pl.BlockSpec((D, NBLK), lambda j: (0, j + B // NBLK)),
            pl.BlockSpec((1, B), lambda j: (0, 0)),
            pl.BlockSpec((NBLK, 1), lambda j: (j, 0)),
            pl.BlockSpec((1, B), lambda j: (0, 0)),
            pl.BlockSpec((NBLK, 1), lambda j: (j, 0)),
            pl.BlockSpec((1, B), lambda j: (0, 0)),
            pl.BlockSpec((NBLK, 1), lambda j: (j, 0)),
        ],
        out_specs=pl.BlockSpec(memory_space=pltpu.SMEM),
        out_shape=jax.ShapeDtypeStruct((1, 1), jnp.float32),
        scratch_shapes=[
            pltpu.VMEM((1, B), jnp.float32),
            pltpu.VMEM((1, B), jnp.float32),
            pltpu.VMEM((1, B), jnp.float32),
        ],
    )(nt2, usert, embt, embt, pos_bias2, samp_bias2, pos_ids2, samp_ids2,
      p_pos2, p_samp2)


def kernel(user_embeddings, item_emb_table, item_bias, positive_item_ids,
           sampled_item_ids, num_tries):
    num_items = item_emb_table.shape[0]
    table_t = item_emb_table.T
    usert = user_embeddings.T
    ids_all = jnp.concatenate([positive_item_ids, sampled_item_ids])
    embt = _tc_gather(table_t, ids_all)
    bias_g = _sc_gather_bias(item_bias, ids_all)

    pos_bias2 = bias_g[:B].reshape(1, B)
    samp_bias2 = bias_g[B:].reshape(S, 1)

    log_range = jnp.log(jnp.float32(num_items + 1.0))
    pf = positive_item_ids.astype(jnp.float32)
    sf = sampled_item_ids.astype(jnp.float32)
    p_pos2 = ((jnp.log(pf + 2.0) - jnp.log(pf + 1.0)) / log_range).reshape(1, B)
    p_samp2 = ((jnp.log(sf + 2.0) - jnp.log(sf + 1.0)) / log_range).reshape(S, 1)

    nt2 = jnp.asarray(num_tries, dtype=jnp.int32).reshape(1, 1)
    loss = _fused_loss(nt2, usert, embt, pos_bias2, samp_bias2,
                       positive_item_ids.reshape(1, B),
                       sampled_item_ids.reshape(S, 1), p_pos2, p_samp2)
    return loss.reshape(())


# speedup vs baseline: 15.8813x; 15.8813x over previous
import functools

import jax
import jax.numpy as jnp
from jax import lax
from jax.experimental import pallas as pl
from jax.experimental.pallas import tpu as pltpu
from jax.experimental.pallas import tpu_sc as plsc

B = 1024
D = 32
S = 10240
N_ALL = B + S
KPG = 128
NW = 32
BPW = N_ALL // NW

NBLK = 1024

_F32_MAX = float(jnp.finfo(jnp.float32).max)
_F32_TINY = float(jnp.finfo(jnp.float32).tiny)


def _gather_body(ids_ref, *refs):
    blocks, out_ref = refs[:KPG], refs[KPG]
    g = pl.program_id(0)
    lane = jax.lax.broadcasted_iota(jnp.int32, (1, 128), 1)
    out = jnp.zeros((D, KPG), jnp.float32)
    for j in range(KPG):
        c = ids_ref[g * KPG + j] & 127
        rolled = pltpu.roll(blocks[j][...], j - c, axis=1)
        out = jnp.where(lane == j, rolled, out)
    off = pl.multiple_of(g * KPG, KPG)
    out_ref[:, pl.ds(off, KPG)] = out


def _tc_gather(table_t, ids):
    block_specs = [
        pl.BlockSpec((D, 128),
                     functools.partial(lambda j, g, ids_ref:
                                       (0, ids_ref[g * KPG + j] >> 7), j))
        for j in range(KPG)
    ]
    grid_spec = pltpu.PrefetchScalarGridSpec(
        num_scalar_prefetch=1,
        grid=(N_ALL // KPG,),
        in_specs=block_specs,
        out_specs=pl.BlockSpec((D, N_ALL), lambda g, ids_ref: (0, 0)),
    )
    return pl.pallas_call(
        _gather_body,
        grid_spec=grid_spec,
        out_shape=jax.ShapeDtypeStruct((D, N_ALL), jnp.float32),
    )(ids, *([table_t] * KPG))


def _sc_gather_bias(bias, ids):
    mesh = plsc.VectorSubcoreMesh(core_axis_name="c", subcore_axis_name="s")

    @functools.partial(
        pl.kernel,
        mesh=mesh,
        compiler_params=pltpu.CompilerParams(use_tc_tiling_on_sc=False),
        out_type=jax.ShapeDtypeStruct((N_ALL,), jnp.float32),
        scratch_types=[
            pltpu.VMEM((BPW,), jnp.int32),
            pltpu.VMEM((BPW,), jnp.float32),
            pltpu.SemaphoreType.DMA,
        ],
    )
    def gather_kernel(bias_hbm, ids_hbm, bias_out, idx_v, bias_v, sem):
        wid = lax.axis_index("s") * 2 + lax.axis_index("c")
        base = wid * BPW
        pltpu.sync_copy(ids_hbm.at[pl.ds(base, BPW)], idx_v)
        pltpu.async_copy(bias_hbm.at[idx_v], bias_v, sem).wait()
        pltpu.sync_copy(bias_v, bias_out.at[pl.ds(base, BPW)])

    return gather_kernel(bias, ids)


def _expected_counts(p, nt):
    acc = jnp.ones_like(p)
    cur = 1.0 - p
    for k in range(32):
        bit = ((nt >> k) & 1).astype(jnp.float32)
        acc = acc * (cur * bit + (1.0 - bit))
        cur = cur * cur
    eq = (nt == S).astype(jnp.float32)
    expected = (p * jnp.float32(S)) * eq + (1.0 - acc) * (1.0 - eq)
    return jnp.maximum(expected, _F32_TINY)


def _flash_body(nt_ref, usert_ref, pos_embt_ref, samp_embt_ref, pos_bias_ref,
                samp_bias_ref, pos_ids_ref, samp_ids_ref, p_pos_ref,
                p_samp_ref, out_ref, m_ref, s_ref, pos_ref):
    j = pl.program_id(0)
    nt = nt_ref[0, 0]

    @pl.when(j == 0)
    def _init():
        e_p = _expected_counts(p_pos_ref[...], nt)
        pos = (jnp.sum(usert_ref[...] * pos_embt_ref[...], axis=0,
                       keepdims=True)
               + pos_bias_ref[...] - jnp.log(e_p))
        pos_ref[...] = pos
        m_ref[...] = pos
        s_ref[...] = jnp.ones_like(pos)

    z = lax.dot_general(samp_embt_ref[...], usert_ref[...],
                        (((0,), (0,)), ((), ())),
                        preferred_element_type=jnp.float32)
    z = z + samp_bias_ref[...]
    hit = samp_ids_ref[...] == pos_ids_ref[...]
    z = jnp.where(hit, -_F32_MAX, z)
    e_s = _expected_counts(p_samp_ref[...], nt)
    z = z - jnp.log(e_s)

    m_old = m_ref[...]
    m_new = jnp.maximum(m_old, jnp.max(z, axis=0, keepdims=True))
    s_ref[...] = (s_ref[...] * jnp.exp(m_old - m_new)
                  + jnp.sum(jnp.exp(z - m_new), axis=0, keepdims=True))
    m_ref[...] = m_new

    @pl.when(j == pl.num_programs(0) - 1)
    def _fin():
        lse = m_ref[...] + jnp.log(s_ref[...])
        out_ref[0, 0] = jnp.sum(lse - pos_ref[...]) / jnp.float32(B)


def _fused_loss(nt2, usert, embt, pos_bias2, samp_bias2, pos_ids2, samp_ids2,
                p_pos2, p_samp2):
    return pl.pallas_call(
        _flash_body,
        grid=(S // NBLK,),
        in_specs=[
            pl.BlockSpec(memory_space=pltpu.SMEM),
            pl.BlockSpec((D, B), lambda j: (0, 0)),
            pl.BlockSpec((D, B), lambda j: (0, 0)),
            pl.BlockSpec((D, NBLK), lambda j: (0, j + B // NBLK)),
            pl.BlockSpec((1, B), lambda j: (0, 0)),
            pl.BlockSpec((NBLK, 1), lambda j: (j, 0)),
            pl.BlockSpec((1, B), lambda j: (0, 0)),
            pl.BlockSpec((NBLK, 1), lambda j: (j, 0)),
            pl.BlockSpec((1, B), lambda j: (0, 0)),
            pl.BlockSpec((NBLK, 1), lambda j: (j, 0)),
        ],
        out_specs=pl.BlockSpec(memory_space=pltpu.SMEM),
        out_shape=jax.ShapeDtypeStruct((1, 1), jnp.float32),
        scratch_shapes=[
            pltpu.VMEM((1, B), jnp.float32),
            pltpu.VMEM((1, B), jnp.float32),
            pltpu.VMEM((1, B), jnp.float32),
        ],
    )(nt2, usert, embt, embt, pos_bias2, samp_bias2, pos_ids2, samp_ids2,
      p_pos2, p_samp2)


def kernel(user_embeddings, item_emb_table, item_bias, positive_item_ids,
           sampled_item_ids, num_tries):
    num_items = item_emb_table.shape[0]
    table_t = item_emb_table.T
    usert = user_embeddings.T
    ids_all = jnp.concatenate([positive_item_ids, sampled_item_ids])
    embt = _tc_gather(table_t, ids_all)
    bias_g = _sc_gather_bias(item_bias, ids_all)

    pos_bias2 = bias_g[:B].reshape(1, B)
    samp_bias2 = bias_g[B:].reshape(S, 1)

    log_range = jnp.log(jnp.float32(num_items + 1.0))
    pf = positive_item_ids.astype(jnp.float32)
    sf = sampled_item_ids.astype(jnp.float32)
    p_pos2 = ((jnp.log(pf + 2.0) - jnp.log(pf + 1.0)) / log_range).reshape(1, B)
    p_samp2 = ((jnp.log(sf + 2.0) - jnp.log(sf + 1.0)) / log_range).reshape(S, 1)

    nt2 = jnp.asarray(num_tries, dtype=jnp.int32).reshape(1, 1)
    loss = _fused_loss(nt2, usert, embt, pos_bias2, samp_bias2,
                       positive_item_ids.reshape(1, B),
                       sampled_item_ids.reshape(S, 1), p_pos2, p_samp2)
    return loss.reshape(())


# speedup vs baseline: 15.9194x; 1.0024x over previous
import functools

import jax
import jax.numpy as jnp
from jax import lax
from jax.experimental import pallas as pl
from jax.experimental.pallas import tpu as pltpu
from jax.experimental.pallas import tpu_sc as plsc

B = 1024
D = 32
S = 10240
N_ALL = B + S
KPG = 256
NW = 32
BPW = N_ALL // NW

NBLK = 1024

_F32_MAX = float(jnp.finfo(jnp.float32).max)
_F32_TINY = float(jnp.finfo(jnp.float32).tiny)


def _gather_body(ids_ref, tcols_ref, *refs):
    del tcols_ref
    blocks, out_ref = refs[:KPG], refs[KPG]
    g = pl.program_id(0)
    lane = jax.lax.broadcasted_iota(jnp.int32, (1, 128), 1)
    halves = [jnp.zeros((D, 128), jnp.float32) for _ in range(KPG // 128)]
    for j in range(KPG):
        c = ids_ref[g * KPG + j] & 127
        jl = j % 128
        rolled = pltpu.roll(blocks[j][...], jl - c, axis=1)
        halves[j // 128] = jnp.where(lane == jl, rolled, halves[j // 128])
    off = pl.multiple_of(g * KPG, KPG)
    for h, half in enumerate(halves):
        out_ref[:, pl.ds(off + h * 128, 128)] = half


def _tc_gather(table_t, ids):
    block_specs = [
        pl.BlockSpec((D, 128),
                     functools.partial(lambda j, g, ids_ref, tcols_ref:
                                       (0, tcols_ref[g * KPG + j]), j))
        for j in range(KPG)
    ]
    grid_spec = pltpu.PrefetchScalarGridSpec(
        num_scalar_prefetch=2,
        grid=(N_ALL // KPG,),
        in_specs=block_specs,
        out_specs=pl.BlockSpec((D, N_ALL), lambda g, ids_ref, tcols_ref: (0, 0)),
    )
    return pl.pallas_call(
        _gather_body,
        grid_spec=grid_spec,
        out_shape=jax.ShapeDtypeStruct((D, N_ALL), jnp.float32),
    )(ids, ids >> 7, *([table_t] * KPG))


def _sc_gather_bias(bias, ids):
    mesh = plsc.VectorSubcoreMesh(core_axis_name="c", subcore_axis_name="s")

    @functools.partial(
        pl.kernel,
        mesh=mesh,
        compiler_params=pltpu.CompilerParams(use_tc_tiling_on_sc=False),
        out_type=jax.ShapeDtypeStruct((N_ALL,), jnp.float32),
        scratch_types=[
            pltpu.VMEM((BPW,), jnp.int32),
            pltpu.VMEM((BPW,), jnp.float32),
            pltpu.SemaphoreType.DMA,
        ],
    )
    def gather_kernel(bias_hbm, ids_hbm, bias_out, idx_v, bias_v, sem):
        wid = lax.axis_index("s") * 2 + lax.axis_index("c")
        base = wid * BPW
        pltpu.sync_copy(ids_hbm.at[pl.ds(base, BPW)], idx_v)
        pltpu.async_copy(bias_hbm.at[idx_v], bias_v, sem).wait()
        pltpu.sync_copy(bias_v, bias_out.at[pl.ds(base, BPW)])

    return gather_kernel(bias, ids)


def _expected_counts(p, nt):
    acc = jnp.ones_like(p)
    cur = 1.0 - p
    for k in range(32):
        bit = ((nt >> k) & 1).astype(jnp.float32)
        acc = acc * (cur * bit + (1.0 - bit))
        cur = cur * cur
    eq = (nt == S).astype(jnp.float32)
    expected = (p * jnp.float32(S)) * eq + (1.0 - acc) * (1.0 - eq)
    return jnp.maximum(expected, _F32_TINY)


def _flash_body(nt_ref, usert_ref, pos_embt_ref, samp_embt_ref, pos_bias_ref,
                samp_bias_ref, pos_ids_ref, samp_ids_ref, p_pos_ref,
                p_samp_ref, out_ref, m_ref, s_ref, pos_ref):
    j = pl.program_id(0)
    nt = nt_ref[0, 0]

    @pl.when(j == 0)
    def _init():
        e_p = _expected_counts(p_pos_ref[...], nt)
        pos = (jnp.sum(usert_ref[...] * pos_embt_ref[...], axis=0,
                       keepdims=True)
               + pos_bias_ref[...] - jnp.log(e_p))
        pos_ref[...] = pos
        m_ref[...] = pos
        s_ref[...] = jnp.ones_like(pos)

    z = lax.dot_general(samp_embt_ref[...], usert_ref[...],
                        (((0,), (0,)), ((), ())),
                        preferred_element_type=jnp.float32)
    z = z + samp_bias_ref[...]
    hit = samp_ids_ref[...] == pos_ids_ref[...]
    z = jnp.where(hit, -_F32_MAX, z)
    e_s = _expected_counts(p_samp_ref[...], nt)
    z = z - jnp.log(e_s)

    m_old = m_ref[...]
    m_new = jnp.maximum(m_old, jnp.max(z, axis=0, keepdims=True))
    s_ref[...] = (s_ref[...] * jnp.exp(m_old - m_new)
                  + jnp.sum(jnp.exp(z - m_new), axis=0, keepdims=True))
    m_ref[...] = m_new

    @pl.when(j == pl.num_programs(0) - 1)
    def _fin():
        lse = m_ref[...] + jnp.log(s_ref[...])
        out_ref[0, 0] = jnp.sum(lse - pos_ref[...]) / jnp.float32(B)


def _fused_loss(nt2, usert, embt, pos_bias2, samp_bias2, pos_ids2, samp_ids2,
                p_pos2, p_samp2):
    return pl.pallas_call(
        _flash_body,
        grid=(S // NBLK,),
        in_specs=[
            pl.BlockSpec(memory_space=pltpu.SMEM),
            pl.BlockSpec((D, B), lambda j: (0, 0)),
            pl.BlockSpec((D, B), lambda j: (0, 0)),
            pl.BlockSpec((D, NBLK), lambda j: (0, j + B // NBLK)),
            pl.BlockSpec((1, B), lambda j: (0, 0)),
            pl.BlockSpec((NBLK, 1), lambda j: (j, 0)),
            pl.BlockSpec((1, B), lambda j: (0, 0)),
            pl.BlockSpec((NBLK, 1), lambda j: (j, 0)),
            pl.BlockSpec((1, B), lambda j: (0, 0)),
            pl.BlockSpec((NBLK, 1), lambda j: (j, 0)),
        ],
        out_specs=pl.BlockSpec(memory_space=pltpu.SMEM),
        out_shape=jax.ShapeDtypeStruct((1, 1), jnp.float32),
        scratch_shapes=[
            pltpu.VMEM((1, B), jnp.float32),
            pltpu.VMEM((1, B), jnp.float32),
            pltpu.VMEM((1, B), jnp.float32),
        ],
    )(nt2, usert, embt, embt, pos_bias2, samp_bias2, pos_ids2, samp_ids2,
      p_pos2, p_samp2)


def kernel(user_embeddings, item_emb_table, item_bias, positive_item_ids,
           sampled_item_ids, num_tries):
    num_items = item_emb_table.shape[0]
    table_t = item_emb_table.T
    usert = user_embeddings.T
    ids_all = jnp.concatenate([positive_item_ids, sampled_item_ids])
    embt = _tc_gather(table_t, ids_all)
    bias_g = _sc_gather_bias(item_bias, ids_all)

    pos_bias2 = bias_g[:B].reshape(1, B)
    samp_bias2 = bias_g[B:].reshape(S, 1)

    log_range = jnp.log(jnp.float32(num_items + 1.0))
    pf = positive_item_ids.astype(jnp.float32)
    sf = sampled_item_ids.astype(jnp.float32)
    p_pos2 = ((jnp.log(pf + 2.0) - jnp.log(pf + 1.0)) / log_range).reshape(1, B)
    p_samp2 = ((jnp.log(sf + 2.0) - jnp.log(sf + 1.0)) / log_range).reshape(S, 1)

    nt2 = jnp.asarray(num_tries, dtype=jnp.int32).reshape(1, 1)
    loss = _fused_loss(nt2, usert, embt, pos_bias2, samp_bias2,
                       positive_item_ids.reshape(1, B),
                       sampled_item_ids.reshape(S, 1), p_pos2, p_samp2)
    return loss.reshape(())
